# baseline (speedup 1.0000x reference)
import jax
import jax.numpy as jnp
import numpy as np
from jax import lax
from jax.experimental import pallas as pl
from jax.experimental.pallas import tpu as pltpu
from jax.experimental.pallas import tpu_sc as plsc

_NUM_OBJ = 4
_NUM_LEVELS = 16
_FPL = 2
_LOG2_T = 19
_BASE_RES = 16
_GROWTH = 1.3819
_T = 1 << _LOG2_T
_N_POINTS = 524288

_P1 = np.int32(np.uint32(2654435761).view(np.int32))
_P2 = np.int32(np.uint32(805459861).view(np.int32))


def _levels():
    meta = []
    off = 0
    for l in range(_NUM_LEVELS):
        res = int(np.floor(_BASE_RES * (_GROWTH ** l)))
        nv = (res + 1) ** 3
        size = min(nv, _T)
        meta.append((res, size, off, size == nv))
        off += size
    return meta, off


_LEVELS, _TOTAL_ROWS = _levels()

_NW = 32
_C = 512
_PER_W = _N_POINTS // _NW
_CHUNKS = _PER_W // _C
_NF = _NUM_LEVELS * _FPL
_NBUF = 3

_SP_LO = 0
_SP_HI = 7
_SP_AL = _LEVELS[_SP_LO][2] - (_LEVELS[_SP_LO][2] % 8)
_SP_LEN = ((_LEVELS[_SP_HI][2] - _SP_AL + 127) // 128) * 128
_SP_SLICE = _SP_LEN // 16


def _body(pos_hbm, tab_hbm, out_hbm,
          pos_v, idx0, idx1, idx2, rows0, rows1, rows2, out_v,
          sp_tab, sem0, sem1, sem2):
    nc = 2
    sid = lax.axis_index("s")
    wid = sid * nc + lax.axis_index("c")
    pt0 = wid * _PER_W

    iota = lax.iota(jnp.int32, 16)
    iota3 = iota * 3

    idx_b = (idx0, idx1, idx2)
    rows_b = (rows0, rows1, rows2)
    sem_b = (sem0, sem1, sem2)

    sp0 = sid * _SP_SLICE
    pltpu.sync_copy(tab_hbm.at[pl.ds(_SP_AL + sp0, _SP_SLICE)],
                    sp_tab.at[pl.ds(sp0, _SP_SLICE)])
    plsc.subcore_barrier()

    def _xyz(g):
        pi = iota3 + g * 48
        x = plsc.load_gather(pos_v, [pi])
        y = plsc.load_gather(pos_v, [pi + 1])
        z = plsc.load_gather(pos_v, [pi + 2])
        return x, y, z

    def phase_a(l):
        res, size, off, dense = _LEVELS[l]
        if _SP_LO <= l < _SP_HI:
            off = off - _SP_AL
        res_f = float(res)
        idx_v = idx_b[l % _NBUF]

        def grp_a(g, carry, off=off, res=res, res_f=res_f, size=size,
                  dense=dense, idx_v=idx_v):
            b = g * 16
            x, y, z = _xyz(g)
            xi = (x * res_f).astype(jnp.int32)
            yi = (y * res_f).astype(jnp.int32)
            zi = (z * res_f).astype(jnp.int32)
            x0 = jnp.minimum(xi, res - 1)
            y0 = jnp.minimum(yi, res - 1)
            z0 = jnp.minimum(zi, res - 1)

            if dense:
                s = res + 1
                hy0 = y0 * s
                hy1 = hy0 + s
                hz0 = z0 * (s * s) + off
                hz1 = hz0 + s * s
            else:
                hy0 = y0 * _P1
                hy1 = hy0 + _P1
                hz0 = z0 * _P2
                hz1 = hz0 + _P2
            hxs = (x0, x0 + 1)
            hys = (hy0, hy1)
            hzs = (hz0, hz1)
            mask = size - 1
            for dx in (0, 1):
                for dy in (0, 1):
                    for dz in (0, 1):
                        k = dx * 4 + dy * 2 + dz
                        if dense:
                            idx = hxs[dx] + hys[dy] + hzs[dz]
                        else:
                            idx = ((hxs[dx] ^ hys[dy] ^ hzs[dz])
                                   & mask) + off
                        idx_v[pl.ds(k * _C + b, 16)] = idx
            return carry

        lax.fori_loop(0, _C // 16, grp_a, 0, unroll=2)

    def _gather_src(l):
        return sp_tab if _SP_LO <= l < _SP_HI else tab_hbm

    def start_gather(l):
        pltpu.make_async_copy(
            _gather_src(l).at[idx_b[l % _NBUF]],
            rows_b[l % _NBUF], sem_b[l % _NBUF]).start()

    def wait_gather(l):
        pltpu.make_async_copy(
            _gather_src(l).at[idx_b[l % _NBUF]],
            rows_b[l % _NBUF], sem_b[l % _NBUF]).wait()

    def phase_c(l):
        res = _LEVELS[l][0]
        res_f = float(res)
        rows_v = rows_b[l % _NBUF]
        cf0 = jnp.full((16,), 2 * l, jnp.int32)
        cf1 = jnp.full((16,), 2 * l + 1, jnp.int32)

        def grp_c(g, carry, res_f=res_f, rows_v=rows_v, cf0=cf0, cf1=cf1):
            b = g * 16
            x, y, z = _xyz(g)
            xf = x * res_f
            yf = y * res_f
            zf = z * res_f
            wx = xf - xf.astype(jnp.int32).astype(jnp.float32)
            wy = yf - yf.astype(jnp.int32).astype(jnp.float32)
            wz = zf - zf.astype(jnp.int32).astype(jnp.float32)
            wx0 = 1.0 - wx
            wy0 = 1.0 - wy
            wz0 = 1.0 - wz
            wxy = (wx0 * wy0, wx0 * wy, wx * wy0, wx * wy)
            wzs = (wz0, wz)
            acc0 = jnp.zeros((16,), jnp.float32)
            acc1 = jnp.zeros((16,), jnp.float32)
            for k in range(8):
                wv = rows_v[pl.ds(k * _C + b, 16)]
                pair = plsc.bitcast(wv, jnp.bfloat16)
                f0, f1 = plsc.unpack(
                    pair, format=plsc.PackFormat.INTERLEAVED)
                w = wxy[k >> 1] * wzs[k & 1]
                acc0 = acc0 + w * f0
                acc1 = acc1 + w * f1
            plsc.store_scatter(out_v, [b + iota, cf0], acc0)
            plsc.store_scatter(out_v, [b + iota, cf1], acc1)
            return carry

        lax.fori_loop(0, _C // 16, grp_c, 0, unroll=2)

    def chunk_body(ci, carry):
        pbase = pt0 + ci * _C
        pltpu.sync_copy(pos_hbm.at[pl.ds(pbase * 3, _C * 3)], pos_v)

        phase_a(0)
        start_gather(0)
        phase_a(1)
        start_gather(1)
        for l in range(_NUM_LEVELS):
            if l + 2 < _NUM_LEVELS:
                phase_a(l + 2)
                start_gather(l + 2)
            wait_gather(l)
            phase_c(l)

        pltpu.sync_copy(out_v, out_hbm.at[pl.ds(pbase, _C)])
        return carry

    lax.fori_loop(0, _CHUNKS, chunk_body, 0, unroll=False)


@jax.jit
def _run(posf, plane):
    mesh = plsc.VectorSubcoreMesh(core_axis_name="c", subcore_axis_name="s")
    f = pl.kernel(
        _body,
        out_type=jax.ShapeDtypeStruct((_N_POINTS, _NF), jnp.float32),
        mesh=mesh,
        compiler_params=pltpu.CompilerParams(
            needs_layout_passes=False, use_tc_tiling_on_sc=False),
        scratch_types=[
            pltpu.VMEM((_C * 3,), jnp.float32),
            pltpu.VMEM((8 * _C,), jnp.int32),
            pltpu.VMEM((8 * _C,), jnp.int32),
            pltpu.VMEM((8 * _C,), jnp.int32),
            pltpu.VMEM((8 * _C,), jnp.int32),
            pltpu.VMEM((8 * _C,), jnp.int32),
            pltpu.VMEM((8 * _C,), jnp.int32),
            pltpu.VMEM((_C, _NF), jnp.float32),
            pltpu.VMEM_SHARED((_SP_LEN,), jnp.int32),
            pltpu.SemaphoreType.DMA,
            pltpu.SemaphoreType.DMA,
            pltpu.SemaphoreType.DMA,
        ],
    )
    return f(posf, plane)


def kernel(positions_flat, obj_id, tables):
    tab = tables[obj_id]
    plane = lax.bitcast_convert_type(
        tab.astype(jnp.bfloat16), jnp.int32)
    posf = positions_flat.reshape(-1)
    return _run(posf, plane)
